# baseline (speedup 1.0000x reference)
import jax
import jax.numpy as jnp
from jax import lax
from jax.experimental import pallas as pl
from jax.experimental.pallas import tpu as pltpu
from jax.experimental.pallas import tpu_sc as plsc

_N = 10000
_BLK = 400
_NB1 = _N // _BLK
_TC_ROWS = 7200
_NB2 = _TC_ROWS // _BLK

_S0 = 7184
_SC_ROWS = _N - _S0
_NTILES = 32
_RPT = _SC_ROWS // _NTILES
_NGRP = _RPT // 8
_COL0 = (0, 2560, 5120, 7680, 9984)
_CLEN = (2560, 2560, 2560, 2304, 16)
_NCHUNK = len(_COL0)


def _pass1_body(x_ref, W1_ref, b1_ref, W2_ref, adj_ref, gt_ref, u_ref, h_ref):
    i = pl.program_id(0)

    @pl.when(i == 0)
    def _():
        u_ref[...] = jnp.dot(x_ref[...], W1_ref[...],
                             preferred_element_type=jnp.float32)

    acc = jnp.dot(adj_ref[...], u_ref[...], preferred_element_type=jnp.float32)
    h_ref[pl.ds(i * _BLK, _BLK), :] = jnp.maximum(acc + b1_ref[...], 0.0)

    @pl.when(i == _NB1 - 1)
    def _():
        gt_ref[...] = lax.dot_general(
            W2_ref[...], h_ref[...], (((0,), (1,)), ((), ())),
            preferred_element_type=jnp.float32)


def _pass2_body(gt_ref, adj_ref, m_ref, acc_ref):
    i = pl.program_id(0)

    @pl.when(i == 0)
    def _():
        acc_ref[...] = jnp.full_like(acc_ref, -jnp.inf)

    part = lax.dot_general(adj_ref[...], gt_ref[...], (((1,), (1,)), ((), ())),
                           preferred_element_type=jnp.float32)
    acc_ref[...] = jnp.maximum(acc_ref[...],
                               jnp.max(part, axis=0, keepdims=True))

    @pl.when(i == _NB2 - 1)
    def _():
        m_ref[...] = acc_ref[...]


def _sc_body(adj_hbm, g_hbm, out_hbm, g_buf, b0, b1, b2, b3, b4, stage,
             s0, s1, s2, s3, s4, so):
    wid = lax.axis_index("s") * 2 + lax.axis_index("c")
    row0 = _S0 + wid * _RPT

    pltpu.sync_copy(g_hbm, g_buf)

    bufs = (b0, b1, b2, b3, b4)
    sems = (s0, s1, s2, s3, s4)

    def copy_obj(grp, cc):
        return pltpu.make_async_copy(
            adj_hbm.at[pl.ds(row0 + grp * 8, 8),
                       pl.ds(_COL0[cc], _CLEN[cc])],
            bufs[cc], sems[cc])

    for cc in range(_NCHUNK):
        copy_obj(0, cc).start()

    zero = jnp.zeros((16,), jnp.float32)

    def grp_body(grp, _):
        accs = [zero] * 16
        for cc in range(_NCHUNK):
            copy_obj(grp, cc).wait()
            buf = bufs[cc]
            c0 = _COL0[cc]

            if _CLEN[cc] > 16:
                @plsc.parallel_loop(0, _CLEN[cc], 16, unroll=2,
                                    carry=tuple(accs))
                def accs_new(p, carry):
                    g0 = g_buf[pl.ds(c0 + p, 16)]
                    g1 = g_buf[pl.ds(_N + c0 + p, 16)]
                    out = []
                    for r in range(8):
                        a = buf[r, pl.ds(p, 16)]
                        out.append(carry[2 * r] + a * g0)
                        out.append(carry[2 * r + 1] + a * g1)
                    return tuple(out)

                accs = list(accs_new)
            else:
                g0 = g_buf[pl.ds(c0, 16)]
                g1 = g_buf[pl.ds(_N + c0, 16)]
                new = []
                for r in range(8):
                    a = buf[r, :]
                    new.append(accs[2 * r] + a * g0)
                    new.append(accs[2 * r + 1] + a * g1)
                accs = new

            @pl.when(grp + 1 < _NGRP)
            def _():
                copy_obj(grp + 1, cc).start()

        for k in range(16):
            stage[pl.ds(k * 16, 16)] = accs[k]
        pltpu.sync_copy(
            stage, out_hbm.at[pl.ds((wid * _RPT + grp * 8) * 32, 256)])
        return 0

    lax.fori_loop(0, _NGRP, grp_body, 0)


def _sc_layer2(adj, g_flat):
    mesh = plsc.VectorSubcoreMesh(core_axis_name="c", subcore_axis_name="s")
    sc = pl.kernel(
        _sc_body,
        out_type=jax.ShapeDtypeStruct((_SC_ROWS * 32,), jnp.float32),
        mesh=mesh,
        scratch_types=[
            pltpu.VMEM((2 * _N,), jnp.float32),
            pltpu.VMEM((8, 2560), jnp.float32),
            pltpu.VMEM((8, 2560), jnp.float32),
            pltpu.VMEM((8, 2560), jnp.float32),
            pltpu.VMEM((8, 2304), jnp.float32),
            pltpu.VMEM((8, 16), jnp.float32),
            pltpu.VMEM((256,), jnp.float32),
            pltpu.SemaphoreType.DMA,
            pltpu.SemaphoreType.DMA,
            pltpu.SemaphoreType.DMA,
            pltpu.SemaphoreType.DMA,
            pltpu.SemaphoreType.DMA,
            pltpu.SemaphoreType.DMA,
        ],
    )
    return sc(adj, g_flat)


def _combine_body(sc_ref, mtc_ref, b2_ref, W3_ref, b3_ref, out_ref):
    a = sc_ref[...]
    s0 = jnp.sum(a[:, 0:16], axis=1)
    s1 = jnp.sum(a[:, 16:32], axis=1)
    m0 = jnp.maximum(jnp.max(s0), mtc_ref[0, 0]) + b2_ref[0, 0]
    m1 = jnp.maximum(jnp.max(s1), mtc_ref[0, 1]) + b2_ref[0, 1]
    val = m0 * W3_ref[0, 0] + m1 * W3_ref[1, 0] + b3_ref[0, 0]
    out_ref[...] = jnp.full((1, 1), val, dtype=jnp.float32)


@jax.jit
def kernel(x, adj, W1, b1, W2, b2, W3, b3):
    n, nfeat = x.shape
    nhid = W1.shape[1]
    nout = W2.shape[1]

    b1r = b1.reshape(1, nhid)
    b2r = b2.reshape(1, nout)
    b3r = b3.reshape(1, 1)

    gt = pl.pallas_call(
        _pass1_body,
        grid=(_NB1,),
        in_specs=[
            pl.BlockSpec((n, nfeat), lambda i: (0, 0)),
            pl.BlockSpec((nfeat, nhid), lambda i: (0, 0)),
            pl.BlockSpec((1, nhid), lambda i: (0, 0)),
            pl.BlockSpec((nhid, nout), lambda i: (0, 0)),
            pl.BlockSpec((_BLK, n), lambda i: (i, 0)),
        ],
        out_specs=pl.BlockSpec((nout, n), lambda i: (0, 0)),
        out_shape=jax.ShapeDtypeStruct((nout, n), jnp.float32),
        scratch_shapes=[
            pltpu.VMEM((n, nhid), jnp.float32),
            pltpu.VMEM((n, nhid), jnp.float32),
        ],
    )(x, W1, b1r, W2, adj)

    sc_sums = _sc_layer2(adj, gt.reshape(-1))

    m_tc = pl.pallas_call(
        _pass2_body,
        grid=(_NB2,),
        in_specs=[
            pl.BlockSpec((nout, n), lambda i: (0, 0)),
            pl.BlockSpec((_BLK, n), lambda i: (i, 0)),
        ],
        out_specs=pl.BlockSpec((1, nout), lambda i: (0, 0)),
        out_shape=jax.ShapeDtypeStruct((1, nout), jnp.float32),
        scratch_shapes=[pltpu.VMEM((1, nout), jnp.float32)],
    )(gt, adj)

    out = pl.pallas_call(
        _combine_body,
        in_specs=[
            pl.BlockSpec((_SC_ROWS, 32), lambda: (0, 0)),
            pl.BlockSpec((1, nout), lambda: (0, 0)),
            pl.BlockSpec((1, nout), lambda: (0, 0)),
            pl.BlockSpec((nout, 1), lambda: (0, 0)),
            pl.BlockSpec((1, 1), lambda: (0, 0)),
        ],
        out_specs=pl.BlockSpec((1, 1), lambda: (0, 0)),
        out_shape=jax.ShapeDtypeStruct((1, 1), jnp.float32),
    )(sc_sums.reshape(_SC_ROWS, 32), m_tc, b2r, W3, b3r)

    return out.reshape(1, 1, 1)
